# baseline (speedup 1.0000x reference)
import functools

import jax
import jax.numpy as jnp
from jax import lax
from jax.experimental import pallas as pl
from jax.experimental.pallas import tpu as pltpu
from jax.experimental.pallas import tpu_sc as plsc

_L = 16


def _sc_add(B, S, D, NC, NS):
    NW = NC * NS
    rows_per_w = S // NW
    R = 32
    nblk = rows_per_w // R
    W = R * D
    mesh = plsc.VectorSubcoreMesh(core_axis_name="c", subcore_axis_name="s")

    @functools.partial(
        pl.kernel,
        mesh=mesh,
        out_type=jax.ShapeDtypeStruct((B * S * D,), jnp.float32),
        scratch_types=[
            pltpu.VMEM((W,), jnp.float32),
            pltpu.VMEM((W,), jnp.float32),
        ],
    )
    def k(x_hbm, tab_hbm, out_hbm, tab_v, io_v):
        wid = lax.axis_index("s") * NC + lax.axis_index("c")
        s0 = wid * rows_per_w

        def block(i, _):
            row0 = s0 + i * R
            pltpu.sync_copy(tab_hbm.at[pl.ds(row0 * D, W)], tab_v)
            for b in range(B):
                base = (b * S + row0) * D
                pltpu.sync_copy(x_hbm.at[pl.ds(base, W)], io_v)

                @plsc.parallel_loop(0, W, step=_L, unroll=8)
                def add_chunk(o):
                    plsc.addupdate(io_v.at[pl.ds(o, _L)], tab_v[pl.ds(o, _L)])

                pltpu.sync_copy(io_v, out_hbm.at[pl.ds(base, W)])
            return 0

        lax.fori_loop(0, nblk, block, 0)

    return k


def kernel(inputs, pos_table):
    B, S, D = inputs.shape
    info = plsc.get_sparse_core_info()
    NC, NS = info.num_cores, info.num_subcores
    table = pos_table[:S].reshape(S * D)
    x = inputs.reshape(B * S * D)
    out = _sc_add(B, S, D, NC, NS)(x, table)
    return out.reshape(B, S, D)


# speedup vs baseline: 2.1317x; 2.1317x over previous
import functools

import jax
import jax.numpy as jnp
from jax import lax
from jax.experimental import pallas as pl
from jax.experimental.pallas import tpu as pltpu
from jax.experimental.pallas import tpu_sc as plsc

_L = 16


def _sc_add(B, S, D, NC, NS):
    NW = NC * NS
    rows_per_w = S // NW
    R = 32
    nblk = rows_per_w // R
    mesh = plsc.VectorSubcoreMesh(core_axis_name="c", subcore_axis_name="s")

    @functools.partial(
        pl.kernel,
        mesh=mesh,
        out_type=jax.ShapeDtypeStruct((B, S, D), jnp.float32),
        scratch_types=[
            pltpu.VMEM((R, D), jnp.float32),
            pltpu.VMEM((R, D), jnp.float32),
        ],
    )
    def k(x_hbm, tab_hbm, out_hbm, tab_v, io_v):
        wid = lax.axis_index("s") * NC + lax.axis_index("c")
        s0 = wid * rows_per_w

        def block(i, _):
            row0 = s0 + i * R
            pltpu.sync_copy(tab_hbm.at[pl.ds(row0, R), :], tab_v)
            for b in range(B):
                pltpu.sync_copy(x_hbm.at[b, pl.ds(row0, R), :], io_v)

                @plsc.parallel_loop(0, R)
                def add_row(r):
                    for c in range(D // _L):
                        sl = pl.ds(c * _L, _L)
                        plsc.addupdate(io_v.at[r, sl], tab_v[r, sl])

                pltpu.sync_copy(io_v, out_hbm.at[b, pl.ds(row0, R), :])
            return 0

        lax.fori_loop(0, nblk, block, 0)

    return k


def kernel(inputs, pos_table):
    B, S, D = inputs.shape
    info = plsc.get_sparse_core_info()
    NC, NS = info.num_cores, info.num_subcores
    return _sc_add(B, S, D, NC, NS)(inputs, pos_table)
